# baseline (speedup 1.0000x reference)
import jax
import jax.numpy as jnp
from jax import lax
from jax.experimental import pallas as pl
from jax.experimental.pallas import tpu as pltpu
from jax.experimental.pallas import tpu_sc as plsc

NUM_CORES = 2
NUM_SUBCORES = 16
LANES = 16
NUM_WORKERS = NUM_CORES * NUM_SUBCORES
BATCH = 16384
BPW = BATCH // NUM_WORKERS
CHUNK = 128
NCHUNK = BPW // CHUNK
D = 32


def _sc_body(user_hbm, item_hbm, q_hbm, p_hbm, bu_hbm, bi_hbm, out_hbm,
             uidx, iidx, qrows, prows, bu_v, bi_v, out_v, sem):
    cid = lax.axis_index("c")
    sid = lax.axis_index("s")
    wid = sid * NUM_CORES + cid
    base = wid * BPW

    for j in range(NCHUNK):
        pltpu.sync_copy(user_hbm.at[pl.ds(base + j * CHUNK, CHUNK)], uidx.at[j])
        pltpu.sync_copy(item_hbm.at[pl.ds(base + j * CHUNK, CHUNK)], iidx.at[j])

    copies = []
    for j in range(NCHUNK):
        dst = pl.ds(j * CHUNK, CHUNK)
        copies.append(pltpu.async_copy(q_hbm.at[uidx.at[j]], qrows.at[dst], sem))
        copies.append(pltpu.async_copy(p_hbm.at[iidx.at[j]], prows.at[dst], sem))
        copies.append(pltpu.async_copy(bu_hbm.at[uidx.at[j]], bu_v.at[dst], sem))
        copies.append(pltpu.async_copy(bi_hbm.at[iidx.at[j]], bi_v.at[dst], sem))
    for c in copies:
        c.wait()

    def chunk_body(c, carry):
        off = pl.multiple_of(c * LANES, LANES)
        row = c * LANES + lax.iota(jnp.int32, LANES)
        acc = jnp.zeros((LANES,), jnp.float32)
        for k in range(D):
            kv = jnp.full((LANES,), k, jnp.int32)
            acc = acc + (plsc.load_gather(qrows, [row, kv]) *
                         plsc.load_gather(prows, [row, kv]))
        out_v[pl.ds(off, LANES)] = acc + bu_v[pl.ds(off, LANES)] + bi_v[pl.ds(off, LANES)]
        return carry

    lax.fori_loop(0, BPW // LANES, chunk_body, 0)

    pltpu.sync_copy(out_v, out_hbm.at[pl.ds(base, BPW)])


_sc_call = pl.kernel(
    _sc_body,
    out_type=jax.ShapeDtypeStruct((BATCH,), jnp.float32),
    mesh=plsc.VectorSubcoreMesh(
        core_axis_name="c", subcore_axis_name="s",
        num_cores=NUM_CORES, num_subcores=NUM_SUBCORES),
    scratch_types=[
        pltpu.VMEM((NCHUNK, CHUNK), jnp.int32),
        pltpu.VMEM((NCHUNK, CHUNK), jnp.int32),
        pltpu.VMEM((BPW, D), jnp.float32),
        pltpu.VMEM((BPW, D), jnp.float32),
        pltpu.VMEM((BPW,), jnp.float32),
        pltpu.VMEM((BPW,), jnp.float32),
        pltpu.VMEM((BPW,), jnp.float32),
        pltpu.SemaphoreType.DMA,
    ],
    compiler_params=pltpu.CompilerParams(
        needs_layout_passes=False, use_tc_tiling_on_sc=False),
)


@jax.jit
def kernel(user_id, item_id, Q, P, b_u, b_i):
    return _sc_call(
        user_id.astype(jnp.int32), item_id.astype(jnp.int32),
        Q, P, b_u.reshape(-1), b_i.reshape(-1))
